# Initial kernel scaffold; baseline (speedup 1.0000x reference)
import jax
import jax.numpy as jnp
from jax.experimental import pallas as pl


def kernel(mem, idx, val):
    raise NotImplementedError("write your pallas kernel here")


# speedup vs baseline: 23.4708x; 23.4708x over previous
import functools

import jax
import jax.numpy as jnp
from jax import lax
from jax.experimental import pallas as pl
from jax.experimental.pallas import tpu as pltpu
from jax.experimental.pallas import tpu_sc as plsc

N_NODES = 1_000_000
B = 16384
D = 64
L = 16
NC = 2
NS = 16
NW = NC * NS
RANGE_BITS = 15
RANGE = 1 << RANGE_BITS
JBITS = 14
JMASK = (1 << JBITS) - 1
NCHUNK = B // L
IDXROWS = 128
RC = 128
SENT = 0x7FFFFFFF


def _dyn_gather(x, i):
    return lax.gather(
        x,
        i[:, None],
        lax.GatherDimensionNumbers(
            offset_dims=(), collapsed_slice_dims=(0,), start_index_map=(0,)
        ),
        (1,),
        mode=lax.GatherScatterMode.PROMISE_IN_BOUNDS,
    )


def _sc_body(idx_hbm, val_hbm, out_hbm, idxv, pbuf, wt, gsm, jsm, rows, sem):
    cid = lax.axis_index("c")
    sid = lax.axis_index("s")
    wid = sid * NC + cid
    iota = lax.iota(jnp.int32, L)

    def outer_body(b, cursor0):
        pltpu.sync_copy(idx_hbm.at[pl.ds(b * IDXROWS, IDXROWS)], idxv)

        def scan_body(r, cursor):
            v = idxv[r]
            m = (v >> RANGE_BITS) == wid
            p = ((v & (RANGE - 1)) << JBITS) | (iota + (b * IDXROWS + r) * L)
            mi = jnp.where(m, 1, 0)
            cnt = plsc.cumsum(mi)
            plsc.store_scatter(pbuf, [cursor + cnt - mi], p, mask=m)
            return cursor + cnt[L - 1]

        return lax.fori_loop(0, IDXROWS, scan_body, cursor0)

    n_w = lax.fori_loop(0, NCHUNK // IDXROWS, outer_body, jnp.int32(0))

    nch = (n_w + (L - 1)) // L
    nrc = (n_w + (RC - 1)) // RC

    perm1 = (iota + 1) & (L - 1)

    def post_body(t, _):
        pk = pbuf[pl.ds(t * L, L)]
        valid = (iota + t * L) < n_w
        pk = jnp.where(valid, pk, SENT)
        ps = jnp.sort(pk)
        nxt = _dyn_gather(ps, perm1)
        kill = ((ps >> JBITS) == (nxt >> JBITS)) & (iota < (L - 1))
        keep = (ps != SENT) & ~kill
        plsc.store_scatter(wt, [ps >> JBITS], ps & JMASK, mask=keep)
        return 0

    lax.fori_loop(0, nch, post_body, 0)

    p0 = pbuf[pl.ds(0, L)]
    pad = jnp.full((L,), p0[0], jnp.int32)

    def dma_body(t, _):
        def fill_body(u, _):
            q = t * (RC // L) + u
            pk = pbuf[pl.ds(q * L, L)]
            valid = (iota + q * L) < n_w
            pk = jnp.where(valid, pk, pad)
            g = plsc.load_gather(wt, [pk >> JBITS])
            gsm[0, pl.ds(u * L, L)] = g
            jsm[0, pl.ds(u * L, L)] = pk & JMASK
            return 0

        lax.fori_loop(0, RC // L, fill_body, 0)
        pltpu.async_copy(val_hbm.at[gsm.at[0]], rows, sem).wait()
        pltpu.async_copy(rows, out_hbm.at[jsm.at[0]], sem).wait()
        return 0

    lax.fori_loop(0, nrc, dma_body, 0)


_sc_call = functools.partial(
    pl.kernel,
    out_type=jax.ShapeDtypeStruct((B, D), jnp.float32),
    mesh=plsc.VectorSubcoreMesh(
        core_axis_name="c", subcore_axis_name="s", num_cores=NC, num_subcores=NS
    ),
    compiler_params=pltpu.CompilerParams(
        needs_layout_passes=False, use_tc_tiling_on_sc=False
    ),
    scratch_types=[
        pltpu.VMEM((IDXROWS, L), jnp.int32),
        pltpu.VMEM((B + L,), jnp.int32),
        pltpu.VMEM((RANGE,), jnp.int32),
        pltpu.VMEM((1, RC), jnp.int32),
        pltpu.VMEM((1, RC), jnp.int32),
        pltpu.VMEM((RC, D), jnp.float32),
        pltpu.SemaphoreType.DMA,
    ],
)(_sc_body)


def kernel(mem, idx, val):
    del mem
    idx32 = jnp.asarray(idx, jnp.int32).reshape(NCHUNK, L)
    return _sc_call(idx32, jnp.asarray(val, jnp.float32))
